# Initial kernel scaffold; baseline (speedup 1.0000x reference)
import jax
import jax.numpy as jnp
from jax.experimental import pallas as pl


def kernel(x, edge_index, W1, b1, W2, b2, Wfc, bfc):
    raise NotImplementedError("write your pallas kernel here")


# speedup vs baseline: 22.5791x; 22.5791x over previous
import functools

import jax
import jax.numpy as jnp
from jax import lax
from jax.experimental import pallas as pl
from jax.experimental.pallas import tpu as pltpu
from jax.experimental.pallas import tpu_sc as plsc

NC = 2
NS = 16
CH = 128


def _pad_up(n, m):
    return ((n + m - 1) // m) * m


def _degree_call(dst2, npad):
    nchunks = dst2.shape[0]
    cpw = nchunks // (NC * NS)
    rpt = npad // NS
    GK = 8
    npc = 8
    prows = rpt // npc

    @functools.partial(
        pl.kernel,
        out_type=jax.ShapeDtypeStruct((NC * npad,), jnp.float32),
        mesh=plsc.VectorSubcoreMesh(core_axis_name="c", subcore_axis_name="s"),
        compiler_params=pltpu.CompilerParams(use_tc_tiling_on_sc=False),
        scratch_types=[
            pltpu.VMEM((GK, CH), jnp.int32),
            pltpu.VMEM((CH,), jnp.float32),
            pltpu.VMEM((prows,), jnp.float32),
            pltpu.VMEM_SHARED((npad,), jnp.float32),
            pltpu.SemaphoreType.DMA,
        ],
    )
    def k(dst_hbm, out_hbm, idx_v, ones_v, z_v, acc_sh, sem):
        c = lax.axis_index("c")
        s = lax.axis_index("s")
        w = c * NS + s

        @pl.loop(0, prows // 16)
        def _(i):
            z_v[pl.ds(i * 16, 16)] = jnp.zeros((16,), jnp.float32)
        for i in range(CH // 16):
            ones_v[pl.ds(i * 16, 16)] = jnp.full((16,), 1.0, jnp.float32)

        @pl.loop(0, npc)
        def _(p):
            pltpu.sync_copy(z_v, acc_sh.at[pl.ds(s * rpt + p * prows, prows)])
        plsc.subcore_barrier()

        @pl.loop(0, cpw // GK)
        def _(gr):
            pltpu.sync_copy(dst_hbm.at[pl.ds(w * cpw + gr * GK, GK)], idx_v)

            @pl.loop(0, GK)
            def _(g):
                pltpu.sync_copy(ones_v, acc_sh.at[idx_v.at[g]], add=True)

        plsc.subcore_barrier()

        @pl.loop(0, npc)
        def _(p):
            r0 = s * rpt + p * prows
            pltpu.sync_copy(acc_sh.at[pl.ds(r0, prows)], z_v)
            pltpu.sync_copy(z_v, out_hbm.at[pl.ds(c * npad + r0, prows)])

    return k(dst2)


def _propagate_call(y, src2, dst2, npad, C):
    nchunks = src2.shape[0]
    cpw = nchunks // (NC * NS)
    rpt = npad // NS
    GK = 8
    npc = 8
    prows = rpt // npc

    @functools.partial(
        pl.kernel,
        out_type=jax.ShapeDtypeStruct((NC, npad, C), jnp.float32),
        mesh=plsc.VectorSubcoreMesh(core_axis_name="c", subcore_axis_name="s"),
        compiler_params=pltpu.CompilerParams(use_tc_tiling_on_sc=False),
        scratch_types=[
            pltpu.VMEM((GK, CH), jnp.int32),
            pltpu.VMEM((GK, CH), jnp.int32),
            pltpu.VMEM((CH, C), jnp.float32),
            pltpu.VMEM((prows, C), jnp.float32),
            pltpu.VMEM_SHARED((npad, C), jnp.float32),
            pltpu.SemaphoreType.DMA,
        ],
    )
    def k(y_hbm, src_hbm, dst_hbm, out_hbm, src_v, dst_v, rows_v, tmp_v,
          acc_sh, sem):
        c = lax.axis_index("c")
        s = lax.axis_index("s")
        w = c * NS + s

        @pl.loop(0, npc)
        def _(p):
            r0 = s * rpt + p * prows
            pltpu.sync_copy(y_hbm.at[pl.ds(r0, prows)], tmp_v)
            pltpu.sync_copy(tmp_v, acc_sh.at[pl.ds(r0, prows)])

        plsc.subcore_barrier()

        @pl.loop(0, cpw // GK)
        def _(gr):
            j0 = w * cpw + gr * GK
            pltpu.sync_copy(src_hbm.at[pl.ds(j0, GK)], src_v)
            pltpu.sync_copy(dst_hbm.at[pl.ds(j0, GK)], dst_v)

            @pl.loop(0, GK)
            def _(g):
                pltpu.async_copy(y_hbm.at[src_v.at[g]], rows_v, sem).wait()
                pltpu.sync_copy(rows_v, acc_sh.at[dst_v.at[g]], add=True)

        plsc.subcore_barrier()

        @pl.loop(0, npc)
        def _(p):
            r0 = s * rpt + p * prows
            pltpu.sync_copy(acc_sh.at[pl.ds(r0, prows)], tmp_v)
            pltpu.sync_copy(tmp_v, out_hbm.at[c, pl.ds(r0, prows)])

    return k(y, src2, dst2)


def _t1_call(degT, xp, npad, bn):
    def body(deg_ref, x_ref, dinv_ref, y1_ref):
        d = jnp.sum(deg_ref[...], axis=1, keepdims=True) + 1.0
        dinv = lax.rsqrt(d)
        dinv_ref[...] = dinv
        y1_ref[...] = x_ref[...] * dinv

    grid = npad // bn
    return pl.pallas_call(
        body,
        grid=(grid,),
        in_specs=[
            pl.BlockSpec((bn, 2), lambda i: (i, 0)),
            pl.BlockSpec((bn, 4), lambda i: (i, 0)),
        ],
        out_specs=[
            pl.BlockSpec((bn, 1), lambda i: (i, 0)),
            pl.BlockSpec((bn, 4), lambda i: (i, 0)),
        ],
        out_shape=[
            jax.ShapeDtypeStruct((npad, 1), jnp.float32),
            jax.ShapeDtypeStruct((npad, 4), jnp.float32),
        ],
    )(degT, xp)


def _t2_call(a0, a1, y1, dinv, W1, b1, W2, npad, bn):
    def body(a0_ref, a1_ref, y1_ref, dinv_ref, w1_ref, b1_ref, w2_ref, y2_ref):
        agg = a0_ref[...] + a1_ref[...] - y1_ref[...]
        dinv = dinv_ref[...]
        z = jnp.dot(agg * dinv, w1_ref[...],
                    preferred_element_type=jnp.float32) + b1_ref[...]
        h1 = jnp.maximum(z, 0.0)
        y2_ref[...] = jnp.dot(h1, w2_ref[...],
                              preferred_element_type=jnp.float32) * dinv

    grid = npad // bn
    return pl.pallas_call(
        body,
        grid=(grid,),
        in_specs=[
            pl.BlockSpec((bn, 4), lambda i: (i, 0)),
            pl.BlockSpec((bn, 4), lambda i: (i, 0)),
            pl.BlockSpec((bn, 4), lambda i: (i, 0)),
            pl.BlockSpec((bn, 1), lambda i: (i, 0)),
            pl.BlockSpec((4, 64), lambda i: (0, 0)),
            pl.BlockSpec((1, 64), lambda i: (0, 0)),
            pl.BlockSpec((64, 32), lambda i: (0, 0)),
        ],
        out_specs=pl.BlockSpec((bn, 32), lambda i: (i, 0)),
        out_shape=jax.ShapeDtypeStruct((npad, 32), jnp.float32),
    )(a0, a1, y1, dinv, W1, b1, W2)


def _t3_call(a0, a1, y2, dinv, b2, Wfc, bfc, n, npad, bn):
    grid = npad // bn

    def body(a0_ref, a1_ref, y2_ref, dinv_ref, b2_ref, wfc_ref, bfc_ref,
             out_ref, acc_ref):
        i = pl.program_id(0)
        agg = a0_ref[...] + a1_ref[...] - y2_ref[...]
        o = jnp.maximum(agg * dinv_ref[...] + b2_ref[...], 0.0)
        row = i * bn + lax.broadcasted_iota(jnp.int32, (bn, 1), 0)
        o = jnp.where(row < n, o, 0.0)
        psum = jnp.sum(o, axis=0, keepdims=True)

        @pl.when(i == 0)
        def _():
            acc_ref[...] = jnp.zeros_like(acc_ref)

        acc_ref[...] += psum

        @pl.when(i == grid - 1)
        def _():
            g = acc_ref[...] / jnp.float32(n)
            logit = jnp.dot(g, wfc_ref[...],
                            preferred_element_type=jnp.float32) + bfc_ref[...]
            out_ref[...] = 1.0 / (1.0 + jnp.exp(-logit))

    return pl.pallas_call(
        body,
        grid=(grid,),
        in_specs=[
            pl.BlockSpec((bn, 32), lambda i: (i, 0)),
            pl.BlockSpec((bn, 32), lambda i: (i, 0)),
            pl.BlockSpec((bn, 32), lambda i: (i, 0)),
            pl.BlockSpec((bn, 1), lambda i: (i, 0)),
            pl.BlockSpec((1, 32), lambda i: (0, 0)),
            pl.BlockSpec((32, 1), lambda i: (0, 0)),
            pl.BlockSpec((1, 1), lambda i: (0, 0)),
        ],
        out_specs=pl.BlockSpec((1, 1), lambda i: (0, 0)),
        out_shape=jax.ShapeDtypeStruct((1, 1), jnp.float32),
        scratch_shapes=[pltpu.VMEM((1, 32), jnp.float32)],
    )(a0, a1, y2, dinv, b2, Wfc, bfc)


def kernel(x, edge_index, W1, b1, W2, b2, Wfc, bfc):
    n = x.shape[0]
    e = edge_index.shape[1]
    npad = _pad_up(n, NC * NS * 8)
    epad = _pad_up(e, CH * NC * NS * 8)
    bn = npad // 16

    src = edge_index[0]
    dst = edge_index[1]
    if epad != e:
        pad = epad - e
        src = jnp.concatenate([src, jnp.zeros((pad,), jnp.int32)])
        dst = jnp.concatenate([dst, jnp.full((pad,), npad - 1, jnp.int32)])
    src2 = src.reshape(-1, CH)
    dst2 = dst.reshape(-1, CH)
    xp = jnp.pad(x, ((0, npad - n), (0, 0)))

    deg = _degree_call(dst2, npad).reshape(NC, npad)
    dinv, y1 = _t1_call(deg.T, xp, npad, bn)
    p1 = _propagate_call(y1, src2, dst2, npad, 4)
    y2 = _t2_call(p1[0], p1[1], y1, dinv, W1,
                  b1.reshape(1, -1), W2, npad, bn)
    p2 = _propagate_call(y2, src2, dst2, npad, 32)
    out = _t3_call(p2[0], p2[1], y2, dinv,
                   b2.reshape(1, -1), Wfc, bfc.reshape(1, 1), n, npad, bn)
    return out.reshape((1,))


# speedup vs baseline: 25.8577x; 1.1452x over previous
import functools

import jax
import jax.numpy as jnp
from jax import lax
from jax.experimental import pallas as pl
from jax.experimental.pallas import tpu as pltpu
from jax.experimental.pallas import tpu_sc as plsc

NC = 2
NS = 16
CH = 128


def _pad_up(n, m):
    return ((n + m - 1) // m) * m


def _pick_prows(rpt, maxrows):
    for p in range(maxrows - maxrows % 16, 15, -16):
        if rpt % p == 0:
            return p
    return 16


def _degree_call(dst2, npad):
    nchunks = dst2.shape[0]
    cpw = nchunks // (NC * NS)
    rpt = npad // NS
    GK = 8
    prows = _pick_prows(rpt, 512)
    npc = rpt // prows

    @functools.partial(
        pl.kernel,
        out_type=jax.ShapeDtypeStruct((NC * npad,), jnp.float32),
        mesh=plsc.VectorSubcoreMesh(core_axis_name="c", subcore_axis_name="s"),
        compiler_params=pltpu.CompilerParams(use_tc_tiling_on_sc=False),
        scratch_types=[
            pltpu.VMEM((GK, CH), jnp.int32),
            pltpu.VMEM((CH,), jnp.float32),
            pltpu.VMEM((prows,), jnp.float32),
            pltpu.VMEM_SHARED((npad,), jnp.float32),
            pltpu.SemaphoreType.DMA,
        ],
    )
    def k(dst_hbm, out_hbm, idx_v, ones_v, z_v, acc_sh, sem):
        c = lax.axis_index("c")
        s = lax.axis_index("s")
        w = c * NS + s

        @pl.loop(0, prows // 16)
        def _(i):
            z_v[pl.ds(i * 16, 16)] = jnp.zeros((16,), jnp.float32)
        for i in range(CH // 16):
            ones_v[pl.ds(i * 16, 16)] = jnp.full((16,), 1.0, jnp.float32)

        @pl.loop(0, npc)
        def _(p):
            pltpu.sync_copy(z_v, acc_sh.at[pl.ds(s * rpt + p * prows, prows)])
        plsc.subcore_barrier()

        @pl.loop(0, cpw // GK)
        def _(gr):
            pltpu.sync_copy(dst_hbm.at[pl.ds(w * cpw + gr * GK, GK)], idx_v)
            sd = [pltpu.async_copy(ones_v, acc_sh.at[idx_v.at[g]], sem,
                                   add=True)
                  for g in range(GK)]
            for d in sd:
                d.wait()

        plsc.subcore_barrier()

        @pl.loop(0, npc)
        def _(p):
            r0 = s * rpt + p * prows
            pltpu.sync_copy(acc_sh.at[pl.ds(r0, prows)], z_v)
            pltpu.sync_copy(z_v, out_hbm.at[pl.ds(c * npad + r0, prows)])

    return k(dst2)


def _propagate_call(y, src2, dst2, npad, C):
    nchunks = src2.shape[0]
    cpw = nchunks // (NC * NS)
    rpt = npad // NS
    GK = 4 if C > 8 else 8
    prows = _pick_prows(rpt, 128 if C > 8 else 512)
    npc = rpt // prows

    @functools.partial(
        pl.kernel,
        out_type=jax.ShapeDtypeStruct((NC, npad, C), jnp.float32),
        mesh=plsc.VectorSubcoreMesh(core_axis_name="c", subcore_axis_name="s"),
        compiler_params=pltpu.CompilerParams(use_tc_tiling_on_sc=False),
        scratch_types=[
            pltpu.VMEM((GK, CH), jnp.int32),
            pltpu.VMEM((GK, CH), jnp.int32),
            pltpu.VMEM((GK, CH, C), jnp.float32),
            pltpu.VMEM((prows, C), jnp.float32),
            pltpu.VMEM_SHARED((npad, C), jnp.float32),
            pltpu.SemaphoreType.DMA,
            pltpu.SemaphoreType.DMA,
        ],
    )
    def k(y_hbm, src_hbm, dst_hbm, out_hbm, src_v, dst_v, rows_v, tmp_v,
          acc_sh, sem, ssem):
        c = lax.axis_index("c")
        s = lax.axis_index("s")
        w = c * NS + s

        @pl.loop(0, npc)
        def _(p):
            r0 = s * rpt + p * prows
            pltpu.sync_copy(y_hbm.at[pl.ds(r0, prows)], tmp_v)
            pltpu.sync_copy(tmp_v, acc_sh.at[pl.ds(r0, prows)])

        plsc.subcore_barrier()

        @pl.loop(0, cpw // GK)
        def _(gr):
            j0 = w * cpw + gr * GK
            pltpu.sync_copy(src_hbm.at[pl.ds(j0, GK)], src_v)
            pltpu.sync_copy(dst_hbm.at[pl.ds(j0, GK)], dst_v)
            gd = [pltpu.async_copy(y_hbm.at[src_v.at[b]], rows_v.at[b], sem)
                  for b in range(GK)]
            sd = []
            for b in range(GK):
                gd[b].wait()
                sd.append(pltpu.async_copy(rows_v.at[b],
                                           acc_sh.at[dst_v.at[b]], ssem,
                                           add=True))
            for d in sd:
                d.wait()

        plsc.subcore_barrier()

        @pl.loop(0, npc)
        def _(p):
            r0 = s * rpt + p * prows
            pltpu.sync_copy(acc_sh.at[pl.ds(r0, prows)], tmp_v)
            pltpu.sync_copy(tmp_v, out_hbm.at[c, pl.ds(r0, prows)])

    return k(y, src2, dst2)


def _t1_call(degT, xp, npad, bn):
    def body(deg_ref, x_ref, dinv_ref, y1_ref):
        d = jnp.sum(deg_ref[...], axis=1, keepdims=True) + 1.0
        dinv = lax.rsqrt(d)
        dinv_ref[...] = dinv
        y1_ref[...] = x_ref[...] * dinv

    grid = npad // bn
    return pl.pallas_call(
        body,
        grid=(grid,),
        in_specs=[
            pl.BlockSpec((bn, 2), lambda i: (i, 0)),
            pl.BlockSpec((bn, 4), lambda i: (i, 0)),
        ],
        out_specs=[
            pl.BlockSpec((bn, 1), lambda i: (i, 0)),
            pl.BlockSpec((bn, 4), lambda i: (i, 0)),
        ],
        out_shape=[
            jax.ShapeDtypeStruct((npad, 1), jnp.float32),
            jax.ShapeDtypeStruct((npad, 4), jnp.float32),
        ],
    )(degT, xp)


def _t2_call(a0, a1, y1, dinv, W1, b1, W2, npad, bn):
    def body(a0_ref, a1_ref, y1_ref, dinv_ref, w1_ref, b1_ref, w2_ref, y2_ref):
        agg = a0_ref[...] + a1_ref[...] - y1_ref[...]
        dinv = dinv_ref[...]
        z = jnp.dot(agg * dinv, w1_ref[...],
                    preferred_element_type=jnp.float32) + b1_ref[...]
        h1 = jnp.maximum(z, 0.0)
        y2_ref[...] = jnp.dot(h1, w2_ref[...],
                              preferred_element_type=jnp.float32) * dinv

    grid = npad // bn
    return pl.pallas_call(
        body,
        grid=(grid,),
        in_specs=[
            pl.BlockSpec((bn, 4), lambda i: (i, 0)),
            pl.BlockSpec((bn, 4), lambda i: (i, 0)),
            pl.BlockSpec((bn, 4), lambda i: (i, 0)),
            pl.BlockSpec((bn, 1), lambda i: (i, 0)),
            pl.BlockSpec((4, 64), lambda i: (0, 0)),
            pl.BlockSpec((1, 64), lambda i: (0, 0)),
            pl.BlockSpec((64, 32), lambda i: (0, 0)),
        ],
        out_specs=pl.BlockSpec((bn, 32), lambda i: (i, 0)),
        out_shape=jax.ShapeDtypeStruct((npad, 32), jnp.float32),
    )(a0, a1, y1, dinv, W1, b1, W2)


def _t3_call(a0, a1, y2, dinv, b2, Wfc, bfc, n, npad, bn):
    grid = npad // bn

    def body(a0_ref, a1_ref, y2_ref, dinv_ref, b2_ref, wfc_ref, bfc_ref,
             out_ref, acc_ref):
        i = pl.program_id(0)
        agg = a0_ref[...] + a1_ref[...] - y2_ref[...]
        o = jnp.maximum(agg * dinv_ref[...] + b2_ref[...], 0.0)
        row = i * bn + lax.broadcasted_iota(jnp.int32, (bn, 1), 0)
        o = jnp.where(row < n, o, 0.0)
        psum = jnp.sum(o, axis=0, keepdims=True)

        @pl.when(i == 0)
        def _():
            acc_ref[...] = jnp.zeros_like(acc_ref)

        acc_ref[...] += psum

        @pl.when(i == grid - 1)
        def _():
            g = acc_ref[...] / jnp.float32(n)
            logit = jnp.dot(g, wfc_ref[...],
                            preferred_element_type=jnp.float32) + bfc_ref[...]
            out_ref[...] = 1.0 / (1.0 + jnp.exp(-logit))

    return pl.pallas_call(
        body,
        grid=(grid,),
        in_specs=[
            pl.BlockSpec((bn, 32), lambda i: (i, 0)),
            pl.BlockSpec((bn, 32), lambda i: (i, 0)),
            pl.BlockSpec((bn, 32), lambda i: (i, 0)),
            pl.BlockSpec((bn, 1), lambda i: (i, 0)),
            pl.BlockSpec((1, 32), lambda i: (0, 0)),
            pl.BlockSpec((32, 1), lambda i: (0, 0)),
            pl.BlockSpec((1, 1), lambda i: (0, 0)),
        ],
        out_specs=pl.BlockSpec((1, 1), lambda i: (0, 0)),
        out_shape=jax.ShapeDtypeStruct((1, 1), jnp.float32),
        scratch_shapes=[pltpu.VMEM((1, 32), jnp.float32)],
    )(a0, a1, y2, dinv, b2, Wfc, bfc)


def kernel(x, edge_index, W1, b1, W2, b2, Wfc, bfc):
    n = x.shape[0]
    e = edge_index.shape[1]
    npad = _pad_up(n, NC * NS * 8)
    epad = _pad_up(e, CH * NC * NS * 8)
    bn = npad // 16

    src = edge_index[0]
    dst = edge_index[1]
    if epad != e:
        pad = epad - e
        src = jnp.concatenate([src, jnp.zeros((pad,), jnp.int32)])
        dst = jnp.concatenate([dst, jnp.full((pad,), npad - 1, jnp.int32)])
    src2 = src.reshape(-1, CH)
    dst2 = dst.reshape(-1, CH)
    xp = jnp.pad(x, ((0, npad - n), (0, 0)))

    deg = _degree_call(dst2, npad).reshape(NC, npad)
    dinv, y1 = _t1_call(deg.T, xp, npad, bn)
    p1 = _propagate_call(y1, src2, dst2, npad, 4)
    y2 = _t2_call(p1[0], p1[1], y1, dinv, W1,
                  b1.reshape(1, -1), W2, npad, bn)
    p2 = _propagate_call(y2, src2, dst2, npad, 32)
    out = _t3_call(p2[0], p2[1], y2, dinv,
                   b2.reshape(1, -1), Wfc, bfc.reshape(1, 1), n, npad, bn)
    return out.reshape((1,))


# speedup vs baseline: 29.3561x; 1.1353x over previous
import functools

import jax
import jax.numpy as jnp
from jax import lax
from jax.experimental import pallas as pl
from jax.experimental.pallas import tpu as pltpu
from jax.experimental.pallas import tpu_sc as plsc

NC = 2
NS = 16
CH = 128


def _pad_up(n, m):
    return ((n + m - 1) // m) * m


def _pick_prows(rpt, maxrows):
    for p in range(maxrows - maxrows % 16, 15, -16):
        if rpt % p == 0:
            return p
    return 16


def _pick_S(cpw, cap):
    for s in range(min(cap, cpw), 7, -1):
        if s % 8 == 0 and cpw % s == 0:
            return s
    return 8


def _degree_call(dst2, npad):
    nchunks = dst2.shape[0]
    cpw = nchunks // (NC * NS)
    rpt = npad // NS
    GK = 8
    prows = _pick_prows(rpt, 512)
    npc = rpt // prows

    @functools.partial(
        pl.kernel,
        out_type=jax.ShapeDtypeStruct((NC * npad,), jnp.float32),
        mesh=plsc.VectorSubcoreMesh(core_axis_name="c", subcore_axis_name="s"),
        compiler_params=pltpu.CompilerParams(use_tc_tiling_on_sc=False),
        scratch_types=[
            pltpu.VMEM((cpw, CH), jnp.int32),
            pltpu.VMEM((CH,), jnp.float32),
            pltpu.VMEM((prows,), jnp.float32),
            pltpu.VMEM_SHARED((npad,), jnp.float32),
            pltpu.SemaphoreType.DMA,
        ],
    )
    def k(dst_hbm, out_hbm, idx_v, ones_v, z_v, acc_sh, sem):
        c = lax.axis_index("c")
        s = lax.axis_index("s")
        w = c * NS + s

        @pl.loop(0, prows // 16)
        def _(i):
            z_v[pl.ds(i * 16, 16)] = jnp.zeros((16,), jnp.float32)
        for i in range(CH // 16):
            ones_v[pl.ds(i * 16, 16)] = jnp.full((16,), 1.0, jnp.float32)

        @pl.loop(0, npc)
        def _(p):
            pltpu.sync_copy(z_v, acc_sh.at[pl.ds(s * rpt + p * prows, prows)])
        pltpu.sync_copy(dst_hbm.at[pl.ds(w * cpw, cpw)], idx_v)
        plsc.subcore_barrier()

        @pl.loop(0, cpw // GK)
        def _(gr):
            for g in range(GK):
                pltpu.async_copy(ones_v, acc_sh.at[idx_v.at[gr * GK + g]],
                                 sem, add=True)

        @pl.loop(0, cpw)
        def _(j):
            pltpu.make_async_copy(out_hbm.at[pl.ds(0, CH)], ones_v, sem).wait()

        plsc.subcore_barrier()

        @pl.loop(0, npc)
        def _(p):
            r0 = s * rpt + p * prows
            pltpu.sync_copy(acc_sh.at[pl.ds(r0, prows)], z_v)
            pltpu.sync_copy(z_v, out_hbm.at[pl.ds(c * npad + r0, prows)])

    return k(dst2)


def _propagate_call(y, src2, dst2, npad, C):
    nchunks = src2.shape[0]
    cpw = nchunks // (NC * NS)
    rpt = npad // NS
    GK = 4 if C > 8 else 8
    S = _pick_S(cpw, 40 if C > 8 else 256)
    prows = _pick_prows(rpt, 64 if C > 8 else 512)
    npc = rpt // prows

    @functools.partial(
        pl.kernel,
        out_type=jax.ShapeDtypeStruct((NC, npad, C), jnp.float32),
        mesh=plsc.VectorSubcoreMesh(core_axis_name="c", subcore_axis_name="s"),
        compiler_params=pltpu.CompilerParams(use_tc_tiling_on_sc=False),
        scratch_types=[
            pltpu.VMEM((S, CH), jnp.int32),
            pltpu.VMEM((S, CH), jnp.int32),
            pltpu.VMEM((GK, CH, C), jnp.float32),
            pltpu.VMEM((prows, C), jnp.float32),
            pltpu.VMEM_SHARED((npad, C), jnp.float32),
            pltpu.SemaphoreType.DMA,
            pltpu.SemaphoreType.DMA,
        ],
    )
    def k(y_hbm, src_hbm, dst_hbm, out_hbm, src_v, dst_v, rows_v, tmp_v,
          acc_sh, sem, ssem):
        c = lax.axis_index("c")
        s = lax.axis_index("s")
        w = c * NS + s

        @pl.loop(0, npc)
        def _(p):
            r0 = s * rpt + p * prows
            pltpu.sync_copy(y_hbm.at[pl.ds(r0, prows)], tmp_v)
            pltpu.sync_copy(tmp_v, acc_sh.at[pl.ds(r0, prows)])

        plsc.subcore_barrier()

        @pl.loop(0, cpw // S)
        def _(sg):
            j0 = w * cpw + sg * S
            pltpu.sync_copy(src_hbm.at[pl.ds(j0, S)], src_v)
            pltpu.sync_copy(dst_hbm.at[pl.ds(j0, S)], dst_v)

            @pl.loop(0, S // GK)
            def _(gr):
                b0 = gr * GK
                gd = [pltpu.async_copy(y_hbm.at[src_v.at[b0 + b]],
                                       rows_v.at[b], sem)
                      for b in range(GK)]
                sd = []
                for b in range(GK):
                    gd[b].wait()
                    sd.append(pltpu.async_copy(rows_v.at[b],
                                               acc_sh.at[dst_v.at[b0 + b]],
                                               ssem, add=True))
                for d in sd:
                    d.wait()

        plsc.subcore_barrier()

        @pl.loop(0, npc)
        def _(p):
            r0 = s * rpt + p * prows
            pltpu.sync_copy(acc_sh.at[pl.ds(r0, prows)], tmp_v)
            pltpu.sync_copy(tmp_v, out_hbm.at[c, pl.ds(r0, prows)])

    return k(y, src2, dst2)


def _t1_call(degT, xp, npad, bn):
    def body(deg_ref, x_ref, dinv_ref, y1_ref):
        d = jnp.sum(deg_ref[...], axis=1, keepdims=True) + 1.0
        dinv = lax.rsqrt(d)
        dinv_ref[...] = dinv
        y1_ref[...] = x_ref[...] * dinv

    grid = npad // bn
    return pl.pallas_call(
        body,
        grid=(grid,),
        in_specs=[
            pl.BlockSpec((bn, 2), lambda i: (i, 0)),
            pl.BlockSpec((bn, 4), lambda i: (i, 0)),
        ],
        out_specs=[
            pl.BlockSpec((bn, 1), lambda i: (i, 0)),
            pl.BlockSpec((bn, 4), lambda i: (i, 0)),
        ],
        out_shape=[
            jax.ShapeDtypeStruct((npad, 1), jnp.float32),
            jax.ShapeDtypeStruct((npad, 4), jnp.float32),
        ],
    )(degT, xp)


def _t2_call(p1, y1, dinv, W1, b1, W2, npad, bn):
    def body(p_ref, y1_ref, dinv_ref, w1_ref, b1_ref, w2_ref, y2_ref):
        agg = p_ref[0] + p_ref[1] - y1_ref[...]
        dinv = dinv_ref[...]
        z = jnp.dot(agg * dinv, w1_ref[...],
                    preferred_element_type=jnp.float32) + b1_ref[...]
        h1 = jnp.maximum(z, 0.0)
        y2_ref[...] = jnp.dot(h1, w2_ref[...],
                              preferred_element_type=jnp.float32) * dinv

    grid = npad // bn
    return pl.pallas_call(
        body,
        grid=(grid,),
        in_specs=[
            pl.BlockSpec((2, bn, 4), lambda i: (0, i, 0)),
            pl.BlockSpec((bn, 4), lambda i: (i, 0)),
            pl.BlockSpec((bn, 1), lambda i: (i, 0)),
            pl.BlockSpec((4, 64), lambda i: (0, 0)),
            pl.BlockSpec((1, 64), lambda i: (0, 0)),
            pl.BlockSpec((64, 32), lambda i: (0, 0)),
        ],
        out_specs=pl.BlockSpec((bn, 32), lambda i: (i, 0)),
        out_shape=jax.ShapeDtypeStruct((npad, 32), jnp.float32),
    )(p1, y1, dinv, W1, b1, W2)


def _t3_call(p2, y2, dinv, b2, Wfc, bfc, n, npad, bn):
    grid = npad // bn

    def body(p_ref, y2_ref, dinv_ref, b2_ref, wfc_ref, bfc_ref,
             out_ref, acc_ref):
        i = pl.program_id(0)
        agg = p_ref[0] + p_ref[1] - y2_ref[...]
        o = jnp.maximum(agg * dinv_ref[...] + b2_ref[...], 0.0)
        row = i * bn + lax.broadcasted_iota(jnp.int32, (bn, 1), 0)
        o = jnp.where(row < n, o, 0.0)
        psum = jnp.sum(o, axis=0, keepdims=True)

        @pl.when(i == 0)
        def _():
            acc_ref[...] = jnp.zeros_like(acc_ref)

        acc_ref[...] += psum

        @pl.when(i == grid - 1)
        def _():
            g = acc_ref[...] / jnp.float32(n)
            logit = jnp.dot(g, wfc_ref[...],
                            preferred_element_type=jnp.float32) + bfc_ref[...]
            out_ref[...] = 1.0 / (1.0 + jnp.exp(-logit))

    return pl.pallas_call(
        body,
        grid=(grid,),
        in_specs=[
            pl.BlockSpec((2, bn, 32), lambda i: (0, i, 0)),
            pl.BlockSpec((bn, 32), lambda i: (i, 0)),
            pl.BlockSpec((bn, 1), lambda i: (i, 0)),
            pl.BlockSpec((1, 32), lambda i: (0, 0)),
            pl.BlockSpec((32, 1), lambda i: (0, 0)),
            pl.BlockSpec((1, 1), lambda i: (0, 0)),
        ],
        out_specs=pl.BlockSpec((1, 1), lambda i: (0, 0)),
        out_shape=jax.ShapeDtypeStruct((1, 1), jnp.float32),
        scratch_shapes=[pltpu.VMEM((1, 32), jnp.float32)],
    )(p2, y2, dinv, b2, Wfc, bfc)


def kernel(x, edge_index, W1, b1, W2, b2, Wfc, bfc):
    n = x.shape[0]
    e = edge_index.shape[1]
    npad = _pad_up(n, NC * NS * 8)
    epad = _pad_up(e, CH * NC * NS * 8)
    bn = npad // 16

    src = edge_index[0]
    dst = edge_index[1]
    if epad != e:
        pad = epad - e
        src = jnp.concatenate([src, jnp.zeros((pad,), jnp.int32)])
        dst = jnp.concatenate([dst, jnp.full((pad,), npad - 1, jnp.int32)])
    src2 = src.reshape(-1, CH)
    dst2 = dst.reshape(-1, CH)
    xp = jnp.pad(x, ((0, npad - n), (0, 0)))

    deg = _degree_call(dst2, npad).reshape(NC, npad)
    dinv, y1 = _t1_call(deg.T, xp, npad, bn)
    p1 = _propagate_call(y1, src2, dst2, npad, 4)
    y2 = _t2_call(p1, y1, dinv, W1,
                  b1.reshape(1, -1), W2, npad, bn)
    p2 = _propagate_call(y2, src2, dst2, npad, 32)
    out = _t3_call(p2, y2, dinv,
                   b2.reshape(1, -1), Wfc, bfc.reshape(1, 1), n, npad, bn)
    return out.reshape((1,))
